# baseline (speedup 1.0000x reference)
import functools

import jax
import jax.numpy as jnp
from jax import lax
from jax.experimental import pallas as pl
from jax.experimental.pallas import tpu as pltpu
from jax.experimental.pallas import tpu_sc as plsc

N = 10000
D = 256
SDIM = 4
PDIM = 64
ODIM = 256
NSEG = 4
SEG = N // NSEG
K = 50
SEGP = 2560
NP = NSEG * SEGP

NC, NS, L = 2, 16, 16
NW = NC * NS
QPW = (N + NW - 1) // NW
KP = 56
CV = SEGP // L
INF = float("inf")

BN = 1000


def _prep_body(x_ref, ws_ref, bs_ref, wp_ref, bp_ref, wst_ref,
               space_ref, prop_ref, p_ref):
    xb = x_ref[...]
    space_ref[...] = xb @ ws_ref[...] + bs_ref[...][None, :]
    prop_ref[...] = xb @ wp_ref[...] + bp_ref[...][None, :]
    ct = lax.dot_general(wst_ref[...], xb, (((1,), (1,)), ((), ())),
                         preferred_element_type=jnp.float32)
    sq = jnp.sum(ct * ct, axis=0, keepdims=True)
    col = lax.broadcasted_iota(jnp.int32, (1, SEGP), 1)
    sq = jnp.where(col < SEG, sq, INF)
    ct_t = ct.astype(jnp.bfloat16).astype(jnp.float32)
    p_ref[...] = jnp.concatenate([ct_t, sq], axis=0)


def _prep(xp, W_s, b_s, W_p, b_p, wst):
    return pl.pallas_call(
        _prep_body,
        grid=(NSEG,),
        in_specs=[
            pl.BlockSpec((SEGP, D), lambda s: (s, 0)),
            pl.BlockSpec((D, SDIM), lambda s: (0, 0)),
            pl.BlockSpec((SDIM,), lambda s: (0,)),
            pl.BlockSpec((D, PDIM), lambda s: (0, 0)),
            pl.BlockSpec((PDIM,), lambda s: (0,)),
            pl.BlockSpec((SDIM, D), lambda s: (0, 0)),
        ],
        out_specs=[
            pl.BlockSpec((SEGP, SDIM), lambda s: (s, 0)),
            pl.BlockSpec((SEGP, PDIM), lambda s: (s, 0)),
            pl.BlockSpec((SDIM + 1, SEGP), lambda s: (0, s)),
        ],
        out_shape=[
            jax.ShapeDtypeStruct((NP, SDIM), jnp.float32),
            jax.ShapeDtypeStruct((NP, PDIM), jnp.float32),
            jax.ShapeDtypeStruct((SDIM + 1, NP), jnp.float32),
        ],
    )(xp, W_s, b_s, W_p, b_p, wst)


def _exchange(ka, ia, kb, ib):
    s = kb < ka
    lo_k = jnp.where(s, kb, ka)
    lo_i = jnp.where(s, ib, ia)
    hi_k = jnp.where(s, ka, kb)
    hi_i = jnp.where(s, ia, ib)
    return lo_k, lo_i, hi_k, hi_i


def _sc_knn(P, prop_pad):
    mesh = plsc.VectorSubcoreMesh(core_axis_name="c", subcore_axis_name="s",
                                  num_cores=NC, num_subcores=NS)

    @functools.partial(
        pl.kernel,
        mesh=mesh,
        compiler_params=pltpu.CompilerParams(needs_layout_passes=False,
                                             use_tc_tiling_on_sc=False),
        out_type=[
            jax.ShapeDtypeStruct((N, KP), jnp.int32),
            jax.ShapeDtypeStruct((N, KP), jnp.float32),
            jax.ShapeDtypeStruct((N, PDIM), jnp.float32),
            jax.ShapeDtypeStruct((N, PDIM), jnp.float32),
        ],
        scratch_types=[
            pltpu.VMEM((NP,), jnp.float32),
            pltpu.VMEM((NP,), jnp.float32),
            pltpu.VMEM((NP,), jnp.float32),
            pltpu.VMEM((NP,), jnp.float32),
            pltpu.VMEM((NP,), jnp.float32),
            pltpu.VMEM((1, 4 * L), jnp.int32),
            pltpu.VMEM((4 * L,), jnp.int32),
            pltpu.VMEM((1, 4 * L), jnp.float32),
            pltpu.VMEM((4 * L,), jnp.float32),
            pltpu.VMEM((4 * L, PDIM), jnp.float32),
            pltpu.VMEM((1, PDIM), jnp.float32),
            pltpu.VMEM((1, PDIM), jnp.float32),
            pltpu.VMEM((3 * L,), jnp.float32),
            pltpu.VMEM((3 * L,), jnp.int32),
            pltpu.VMEM((4 * L,), jnp.float32),
            pltpu.VMEM((4 * L,), jnp.int32),
            pltpu.VMEM((L,), jnp.float32),
            pltpu.SMEM((1,), jnp.int32),
            pltpu.SemaphoreType.DMA,
            pltpu.SemaphoreType.DMA,
        ],
    )
    def body(p_hbm, prop_hbm, idx_hbm, dsq_hbm, fmean_hbm, fmax_hbm,
             p0, p1, p2, p3, psq, ibuf, ibufp, dbuf, wbuf, gbuf,
             mbuf, xbuf, pend_k, pend_i, tkb, tib, taub, pcr, sem, osem):
        wid = lax.axis_index("s") * NC + lax.axis_index("c")
        pltpu.sync_copy(p_hbm.at[pl.ds(0 * NP, NP)], p0)
        pltpu.sync_copy(p_hbm.at[pl.ds(1 * NP, NP)], p1)
        pltpu.sync_copy(p_hbm.at[pl.ds(2 * NP, NP)], p2)
        pltpu.sync_copy(p_hbm.at[pl.ds(3 * NP, NP)], p3)
        pltpu.sync_copy(p_hbm.at[pl.ds(4 * NP, NP)], psq)
        lane = lax.iota(jnp.int32, L)

        def qloop(i, carry_q):
            q = wid * QPW + i

            @pl.when(q < N)
            def _():
                seg = q // SEG
                segbase = seg * SEGP
                qpos = segbase + (q - seg * SEG)
                qsplat = jnp.full((L,), qpos, jnp.int32)
                xq0 = plsc.load_gather(p0, [qsplat])
                xq1 = plsc.load_gather(p1, [qsplat])
                xq2 = plsc.load_gather(p2, [qsplat])
                xq3 = plsc.load_gather(p3, [qsplat])
                sqq = plsc.load_gather(psq, [qsplat])

                def _key_at(base):
                    jx0 = p0[pl.ds(base, L)]
                    jx1 = p1[pl.ds(base, L)]
                    jx2 = p2[pl.ds(base, L)]
                    jx3 = p3[pl.ds(base, L)]
                    jsq = psq[pl.ds(base, L)]
                    dot = (jx0 * xq0 + jx1 * xq1) + (jx2 * xq2 + jx3 * xq3)
                    return jsq - (dot + dot)

                def merge32(cc32, st):
                    t0, t1, t2, t3, i0, i1, i2, i3 = st
                    base = segbase + cc32 * (2 * L)
                    k1 = _key_at(base)
                    k2 = _key_at(base + L)
                    l1 = cc32 * (2 * L) + lane
                    l2 = l1 + L
                    a, ai = plsc.sort_key_val(k1, l1, descending=True)
                    b, bi = plsc.sort_key_val(k2, l2)
                    lo, loi, hi, hii = _exchange(a, ai, b, bi)
                    hd, hdi = plsc.sort_key_val(hi, hii, descending=True)
                    ld, ldi = plsc.sort_key_val(lo, loi, descending=True)
                    s2 = hd < t2
                    t2n = jnp.where(s2, hd, t2)
                    i2n = jnp.where(s2, hdi, i2)
                    s3 = ld < t3
                    t3n = jnp.where(s3, ld, t3)
                    i3n = jnp.where(s3, ldi, i3)
                    a0, ja0, b0, jb0 = _exchange(t0, i0, t2n, i2n)
                    a1, ja1, b1, jb1 = _exchange(t1, i1, t3n, i3n)
                    c0, jc0, c1, jc1 = _exchange(a0, ja0, a1, ja1)
                    d0, jd0, d1, jd1 = _exchange(b0, jb0, b1, jb1)
                    f0, g0 = plsc.sort_key_val(c0, jc0)
                    f1, g1 = plsc.sort_key_val(c1, jc1)
                    f2, g2 = plsc.sort_key_val(d0, jd0)
                    f3, g3 = plsc.sort_key_val(d1, jd1)
                    return f0, f1, f2, f3, g0, g1, g2, g3

                def cbody(c, st):
                    st = merge32(c * 2, st)
                    return merge32(c * 2 + 1, st)

                finf = jnp.full((L,), INF, jnp.float32)
                zi = jnp.zeros((L,), jnp.int32)
                st0 = (finf, finf, finf, finf, zi, zi, zi, zi)
                res = lax.fori_loop(0, CV // 4, cbody, st0)
                t_vecs = res[0:4]
                i_vecs = res[4:8]

                @pl.when(i > 0)
                def _():
                    for cd in _out_copies(q - 1):
                        cd.wait()

                for r in range(4):
                    ibufp[pl.ds(r * L, L)] = i_vecs[r] + segbase
                gcopy = pltpu.async_copy(prop_hbm.at[ibufp], gbuf, sem)
                for r in range(4):
                    dsq = jnp.maximum(t_vecs[r] + sqq, 0.0)
                    w = jnp.exp(dsq * -10.0)
                    dbuf[0, pl.ds(r * L, L)] = dsq
                    wbuf[pl.ds(r * L, L)] = w
                    ibuf[0, pl.ds(r * L, L)] = i_vecs[r] + seg * SEG
                gcopy.wait()

                UA = 5

                def abody(kk, acc):
                    a = list(acc)
                    for uu in range(UA):
                        k = kk * UA + uu
                        wk = plsc.load_gather(
                            wbuf, [jnp.full((L,), k, jnp.int32)])
                        for j in range(4):
                            row = gbuf[k, pl.ds(j * L, L)] * wk
                            a[j] = a[j] + row
                            a[4 + j] = jnp.maximum(a[4 + j], row)
                    return tuple(a)

                zf = jnp.zeros((L,), jnp.float32)
                ninf = jnp.full((L,), -INF, jnp.float32)
                acc = lax.fori_loop(0, K // UA, abody,
                                    (zf, zf, zf, zf, ninf, ninf, ninf, ninf))
                for j in range(4):
                    mbuf[0, pl.ds(j * L, L)] = acc[j] * jnp.float32(1.0 / K)
                    xbuf[0, pl.ds(j * L, L)] = acc[4 + j]

                for cd in _out_copies(q):
                    cd.start()

            return carry_q

        def _out_copies(qq):
            return (
                pltpu.make_async_copy(ibuf.at[:, pl.ds(0, KP)],
                                      idx_hbm.at[pl.ds(qq, 1)], osem),
                pltpu.make_async_copy(dbuf.at[:, pl.ds(0, KP)],
                                      dsq_hbm.at[pl.ds(qq, 1)], osem),
                pltpu.make_async_copy(mbuf, fmean_hbm.at[pl.ds(qq, 1)], osem),
                pltpu.make_async_copy(xbuf, fmax_hbm.at[pl.ds(qq, 1)], osem),
            )

        lax.fori_loop(0, QPW, qloop, 0)
        nvalid = jnp.minimum(QPW, N - wid * QPW)
        for cd in _out_copies(wid * QPW + nvalid - 1):
            cd.wait()

    return body(P, prop_pad)


def _out_body(x_ref, fmean_ref, fmax_ref, wo_ref, bo_ref, out_ref):
    wo = wo_ref[...]
    acc = x_ref[...] @ wo[:D, :]
    acc += fmean_ref[...] @ wo[D:D + PDIM, :]
    acc += fmax_ref[...] @ wo[D + PDIM:, :]
    out_ref[...] = jnp.maximum(acc + bo_ref[...][None, :], 0.0)


def _out_proj(x, fmean, fmax, W_o, b_o):
    return pl.pallas_call(
        _out_body,
        grid=(N // BN,),
        in_specs=[
            pl.BlockSpec((BN, D), lambda i: (i, 0)),
            pl.BlockSpec((BN, PDIM), lambda i: (i, 0)),
            pl.BlockSpec((BN, PDIM), lambda i: (i, 0)),
            pl.BlockSpec((D + 2 * PDIM, ODIM), lambda i: (0, 0)),
            pl.BlockSpec((ODIM,), lambda i: (0,)),
        ],
        out_specs=pl.BlockSpec((BN, ODIM), lambda i: (i, 0)),
        out_shape=jax.ShapeDtypeStruct((N, ODIM), jnp.float32),
    )(x, fmean, fmax, W_o, b_o)


def kernel(x, row_splits, W_s, b_s, W_p, b_p, W_o, b_o):
    xp = jnp.pad(x.reshape(NSEG, SEG, D),
                 ((0, 0), (0, SEGP - SEG), (0, 0))).reshape(NP, D)
    space_pad, prop_pad, P = _prep(xp, W_s, b_s, W_p, b_p, W_s.T)
    nbr, dsq, fmean, fmax = _sc_knn(P.reshape(-1), prop_pad)
    nbr = nbr[:, :K]
    dsq = dsq[:, :K]
    space = space_pad.reshape(NSEG, SEGP, SDIM)[:, :SEG].reshape(N, SDIM)
    out = _out_proj(x, fmean, fmax, W_o, b_o)
    return (out, nbr, dsq, space)


# speedup vs baseline: 1.0513x; 1.0513x over previous
import functools

import jax
import jax.numpy as jnp
from jax import lax
from jax.experimental import pallas as pl
from jax.experimental.pallas import tpu as pltpu
from jax.experimental.pallas import tpu_sc as plsc

N = 10000
D = 256
SDIM = 4
PDIM = 64
ODIM = 256
NSEG = 4
SEG = N // NSEG
K = 50
SEGP = 2560
NP = NSEG * SEGP

NC, NS, L = 2, 16, 16
NW = NC * NS
QPW = (N + NW - 1) // NW
KP = 56
CV = SEGP // L
INF = float("inf")

BN = 1000


def _prep_body(x_ref, ws_ref, bs_ref, wp_ref, bp_ref, wst_ref,
               space_ref, prop_ref, p_ref):
    xb = x_ref[...]
    space_ref[...] = xb @ ws_ref[...] + bs_ref[...][None, :]
    prop_ref[...] = xb @ wp_ref[...] + bp_ref[...][None, :]
    ct = lax.dot_general(wst_ref[...], xb, (((1,), (1,)), ((), ())),
                         preferred_element_type=jnp.float32)
    sq = jnp.sum(ct * ct, axis=0, keepdims=True)
    col = lax.broadcasted_iota(jnp.int32, (1, SEGP), 1)
    sq = jnp.where(col < SEG, sq, INF)
    ct_t = ct.astype(jnp.bfloat16).astype(jnp.float32)
    p_ref[...] = jnp.concatenate([ct_t, sq], axis=0)


def _prep(xp, W_s, b_s, W_p, b_p, wst):
    return pl.pallas_call(
        _prep_body,
        grid=(NSEG,),
        in_specs=[
            pl.BlockSpec((SEGP, D), lambda s: (s, 0)),
            pl.BlockSpec((D, SDIM), lambda s: (0, 0)),
            pl.BlockSpec((SDIM,), lambda s: (0,)),
            pl.BlockSpec((D, PDIM), lambda s: (0, 0)),
            pl.BlockSpec((PDIM,), lambda s: (0,)),
            pl.BlockSpec((SDIM, D), lambda s: (0, 0)),
        ],
        out_specs=[
            pl.BlockSpec((SEGP, SDIM), lambda s: (s, 0)),
            pl.BlockSpec((SEGP, PDIM), lambda s: (s, 0)),
            pl.BlockSpec((SDIM + 1, SEGP), lambda s: (0, s)),
        ],
        out_shape=[
            jax.ShapeDtypeStruct((NP, SDIM), jnp.float32),
            jax.ShapeDtypeStruct((NP, PDIM), jnp.float32),
            jax.ShapeDtypeStruct((SDIM + 1, NP), jnp.float32),
        ],
    )(xp, W_s, b_s, W_p, b_p, wst)


def _exchange(ka, ia, kb, ib):
    s = kb < ka
    lo_k = jnp.where(s, kb, ka)
    lo_i = jnp.where(s, ib, ia)
    hi_k = jnp.where(s, ka, kb)
    hi_i = jnp.where(s, ia, ib)
    return lo_k, lo_i, hi_k, hi_i


def _sc_knn(P, prop_pad):
    mesh = plsc.VectorSubcoreMesh(core_axis_name="c", subcore_axis_name="s",
                                  num_cores=NC, num_subcores=NS)

    @functools.partial(
        pl.kernel,
        mesh=mesh,
        compiler_params=pltpu.CompilerParams(needs_layout_passes=False,
                                             use_tc_tiling_on_sc=False),
        out_type=[
            jax.ShapeDtypeStruct((N, KP), jnp.int32),
            jax.ShapeDtypeStruct((N, KP), jnp.float32),
            jax.ShapeDtypeStruct((N, PDIM), jnp.float32),
            jax.ShapeDtypeStruct((N, PDIM), jnp.float32),
        ],
        scratch_types=[
            pltpu.VMEM((NP,), jnp.float32),
            pltpu.VMEM((NP,), jnp.float32),
            pltpu.VMEM((NP,), jnp.float32),
            pltpu.VMEM((NP,), jnp.float32),
            pltpu.VMEM((NP,), jnp.float32),
            pltpu.VMEM((1, 4 * L), jnp.int32),
            pltpu.VMEM((4 * L,), jnp.int32),
            pltpu.VMEM((1, 4 * L), jnp.float32),
            pltpu.VMEM((4 * L,), jnp.float32),
            pltpu.VMEM((4 * L, PDIM), jnp.float32),
            pltpu.VMEM((1, PDIM), jnp.float32),
            pltpu.VMEM((1, PDIM), jnp.float32),
            pltpu.SemaphoreType.DMA,
            pltpu.SemaphoreType.DMA,
        ],
    )
    def body(p_hbm, prop_hbm, idx_hbm, dsq_hbm, fmean_hbm, fmax_hbm,
             p0, p1, p2, p3, psq, ibuf, ibufp, dbuf, wbuf, gbuf,
             mbuf, xbuf, sem, osem):
        wid = lax.axis_index("s") * NC + lax.axis_index("c")
        pltpu.sync_copy(p_hbm.at[pl.ds(0 * NP, NP)], p0)
        pltpu.sync_copy(p_hbm.at[pl.ds(1 * NP, NP)], p1)
        pltpu.sync_copy(p_hbm.at[pl.ds(2 * NP, NP)], p2)
        pltpu.sync_copy(p_hbm.at[pl.ds(3 * NP, NP)], p3)
        pltpu.sync_copy(p_hbm.at[pl.ds(4 * NP, NP)], psq)
        lane = lax.iota(jnp.int32, L)

        def qloop(i, carry_q):
            q = wid * QPW + i

            @pl.when(q < N)
            def _():
                seg = q // SEG
                segbase = seg * SEGP
                qpos = segbase + (q - seg * SEG)
                qsplat = jnp.full((L,), qpos, jnp.int32)
                xq0 = plsc.load_gather(p0, [qsplat])
                xq1 = plsc.load_gather(p1, [qsplat])
                xq2 = plsc.load_gather(p2, [qsplat])
                xq3 = plsc.load_gather(p3, [qsplat])
                sqq = plsc.load_gather(psq, [qsplat])

                def _key_at(base):
                    jx0 = p0[pl.ds(base, L)]
                    jx1 = p1[pl.ds(base, L)]
                    jx2 = p2[pl.ds(base, L)]
                    jx3 = p3[pl.ds(base, L)]
                    jsq = psq[pl.ds(base, L)]
                    dot = (jx0 * xq0 + jx1 * xq1) + (jx2 * xq2 + jx3 * xq3)
                    return jsq - (dot + dot)

                def cbody(c, st):
                    t0, t1, t2, t3, i0, i1, i2, i3 = st
                    base = segbase + c * (2 * L)
                    k1 = _key_at(base)
                    k2 = _key_at(base + L)
                    l1 = c * (2 * L) + lane
                    l2 = l1 + L
                    a, ai = plsc.sort_key_val(k1, l1, descending=True)
                    b, bi = plsc.sort_key_val(k2, l2)
                    lo, loi, hi, hii = _exchange(a, ai, b, bi)
                    hd, hdi = plsc.sort_key_val(hi, hii, descending=True)
                    ld, ldi = plsc.sort_key_val(lo, loi, descending=True)
                    s2 = hd < t2
                    t2n = jnp.where(s2, hd, t2)
                    i2n = jnp.where(s2, hdi, i2)
                    s3 = ld < t3
                    t3n = jnp.where(s3, ld, t3)
                    i3n = jnp.where(s3, ldi, i3)
                    a0, ja0, b0, jb0 = _exchange(t0, i0, t2n, i2n)
                    a1, ja1, b1, jb1 = _exchange(t1, i1, t3n, i3n)
                    c0, jc0, c1, jc1 = _exchange(a0, ja0, a1, ja1)
                    d0, jd0, d1, jd1 = _exchange(b0, jb0, b1, jb1)
                    f0, g0 = plsc.sort_key_val(c0, jc0)
                    f1, g1 = plsc.sort_key_val(c1, jc1)
                    f2, g2 = plsc.sort_key_val(d0, jd0)
                    f3, g3 = plsc.sort_key_val(d1, jd1)
                    return f0, f1, f2, f3, g0, g1, g2, g3

                finf = jnp.full((L,), INF, jnp.float32)
                zi = jnp.zeros((L,), jnp.int32)
                st0 = (finf, finf, finf, finf, zi, zi, zi, zi)
                res = lax.fori_loop(0, CV // 2, cbody, st0)
                t_vecs = res[0:4]
                i_vecs = res[4:8]

                @pl.when(i > 0)
                def _():
                    for cd in _out_copies(q - 1):
                        cd.wait()

                for r in range(4):
                    ibufp[pl.ds(r * L, L)] = i_vecs[r] + segbase
                gcopy = pltpu.async_copy(prop_hbm.at[ibufp], gbuf, sem)
                for r in range(4):
                    dsq = jnp.maximum(t_vecs[r] + sqq, 0.0)
                    w = jnp.exp(dsq * -10.0)
                    dbuf[0, pl.ds(r * L, L)] = dsq
                    wbuf[pl.ds(r * L, L)] = w
                    ibuf[0, pl.ds(r * L, L)] = i_vecs[r] + seg * SEG
                gcopy.wait()

                UA = 5

                def abody(kk, acc):
                    a = list(acc)
                    for uu in range(UA):
                        k = kk * UA + uu
                        wk = plsc.load_gather(
                            wbuf, [jnp.full((L,), k, jnp.int32)])
                        for j in range(4):
                            row = gbuf[k, pl.ds(j * L, L)] * wk
                            a[j] = a[j] + row
                            a[4 + j] = jnp.maximum(a[4 + j], row)
                    return tuple(a)

                zf = jnp.zeros((L,), jnp.float32)
                ninf = jnp.full((L,), -INF, jnp.float32)
                acc = lax.fori_loop(0, K // UA, abody,
                                    (zf, zf, zf, zf, ninf, ninf, ninf, ninf))
                for j in range(4):
                    mbuf[0, pl.ds(j * L, L)] = acc[j] * jnp.float32(1.0 / K)
                    xbuf[0, pl.ds(j * L, L)] = acc[4 + j]

                for cd in _out_copies(q):
                    cd.start()

            return carry_q

        def _out_copies(qq):
            return (
                pltpu.make_async_copy(ibuf.at[:, pl.ds(0, KP)],
                                      idx_hbm.at[pl.ds(qq, 1)], osem),
                pltpu.make_async_copy(dbuf.at[:, pl.ds(0, KP)],
                                      dsq_hbm.at[pl.ds(qq, 1)], osem),
                pltpu.make_async_copy(mbuf, fmean_hbm.at[pl.ds(qq, 1)], osem),
                pltpu.make_async_copy(xbuf, fmax_hbm.at[pl.ds(qq, 1)], osem),
            )

        lax.fori_loop(0, QPW, qloop, 0)
        nvalid = jnp.minimum(QPW, N - wid * QPW)
        for cd in _out_copies(wid * QPW + nvalid - 1):
            cd.wait()

    return body(P, prop_pad)


def _out_body(x_ref, fmean_ref, fmax_ref, wo_ref, bo_ref, out_ref):
    wo = wo_ref[...]
    acc = x_ref[...] @ wo[:D, :]
    acc += fmean_ref[...] @ wo[D:D + PDIM, :]
    acc += fmax_ref[...] @ wo[D + PDIM:, :]
    out_ref[...] = jnp.maximum(acc + bo_ref[...][None, :], 0.0)


def _out_proj(x, fmean, fmax, W_o, b_o):
    return pl.pallas_call(
        _out_body,
        grid=(N // BN,),
        in_specs=[
            pl.BlockSpec((BN, D), lambda i: (i, 0)),
            pl.BlockSpec((BN, PDIM), lambda i: (i, 0)),
            pl.BlockSpec((BN, PDIM), lambda i: (i, 0)),
            pl.BlockSpec((D + 2 * PDIM, ODIM), lambda i: (0, 0)),
            pl.BlockSpec((ODIM,), lambda i: (0,)),
        ],
        out_specs=pl.BlockSpec((BN, ODIM), lambda i: (i, 0)),
        out_shape=jax.ShapeDtypeStruct((N, ODIM), jnp.float32),
    )(x, fmean, fmax, W_o, b_o)


def kernel(x, row_splits, W_s, b_s, W_p, b_p, W_o, b_o):
    xp = jnp.pad(x.reshape(NSEG, SEG, D),
                 ((0, 0), (0, SEGP - SEG), (0, 0))).reshape(NP, D)
    space_pad, prop_pad, P = _prep(xp, W_s, b_s, W_p, b_p, W_s.T)
    nbr, dsq, fmean, fmax = _sc_knn(P.reshape(-1), prop_pad)
    nbr = nbr[:, :K]
    dsq = dsq[:, :K]
    space = space_pad.reshape(NSEG, SEGP, SDIM)[:, :SEG].reshape(N, SDIM)
    out = _out_proj(x, fmean, fmax, W_o, b_o)
    return (out, nbr, dsq, space)
